# baseline (speedup 1.0000x reference)
import functools

import jax
import jax.numpy as jnp
from jax import lax
from jax.experimental import pallas as pl
from jax.experimental.pallas import tpu as pltpu
from jax.experimental.pallas import tpu_sc as plsc

_ND = 50
_HWD = 10
_VOCAB = 101
_L = 16
_F32 = jnp.float32
_I32 = jnp.int32

_OFF_LAYER, _OFF_HEAD, _OFF_MLP, _OFF_EMBED, _OFF_BIAS = 0, 16, 80, 128, 144


def _full(v):
    return jnp.full((_L,), v, _I32)


def _rne_bf16(v):
    b = plsc.bitcast(v, _I32)
    r = b + jnp.int32(0x7FFF) + ((b >> 16) & 1)
    return plsc.bitcast(r & jnp.int32(-65536), _F32)


def _similarity(xv, hwv, hwtv, iota):
    idxb = (xv[...] * 101.0).astype(_I32)
    dcl = [jnp.minimum(iota + 16 * j, _ND - 1) for j in range(4)]
    gvs = [dc * _VOCAB + idxb for dc in dcl]

    def dot_step(k, accs):
        hwk = _rne_bf16(plsc.load_gather(hwv, [iota * 0 + (k + 1)]))
        return tuple(
            accs[j]
            + hwk * _rne_bf16(plsc.load_gather(hwtv, [dcl[j] * _HWD + k]))
            for j in range(4)
        )

    zero = jnp.zeros((_L,), _F32)
    accs = lax.fori_loop(0, _HWD, dot_step, (zero, zero, zero, zero))
    scale = jnp.float32(1.0 / (_HWD ** 0.5))
    zs = [a * scale for a in accs]
    m = jnp.max(jnp.maximum(jnp.maximum(zs[0], zs[1]),
                            jnp.maximum(zs[2], zs[3])))
    es = [jnp.where(iota + 16 * j < _ND, jnp.exp(zs[j] - m),
                    jnp.float32(0.0)) for j in range(4)]
    s = jnp.sum(es[0] + es[1] + es[2] + es[3])
    rv = jnp.full((_L,), 1.0, _F32) / jnp.broadcast_to(s, (_L,))
    return [e * rv for e in es], gvs


def _wide_group(tbl_hbm, gidx, rows_ref, sim_ref, sims, iota, t,
                ov_ref, o_all, off, sem):
    cp = pltpu.async_copy(tbl_hbm.at[gidx], rows_ref, sem)
    for j in range(4):
        sim_ref[pl.ds(16 + 16 * j, 16)] = sims[j]
    cp.wait()
    cols = iota + 16 * t

    def step(d, acc):
        sd = _rne_bf16(plsc.load_gather(sim_ref, [iota * 0 + (16 + d)]))
        rq = _rne_bf16(plsc.load_gather(rows_ref, [_full(0) + d, cols]))
        return acc + sd * rq

    ov_ref[...] = lax.fori_loop(0, _ND, step, jnp.zeros((_L,), _F32))
    pltpu.sync_copy(ov_ref, o_all.at[pl.ds(off + 16 * t, 16)])


def _narrow(tbl_hbm, tbl_ref, gvs, sims, iota, ov_ref, o_all, off, nch, sem):
    cp = pltpu.async_copy(tbl_hbm, tbl_ref, sem)
    simq = [_rne_bf16(sj) for sj in sims]
    cp.wait()
    ov = jnp.zeros((_L,), _F32)
    for c in range(nch):
        acc = jnp.zeros((_L,), _F32)
        for j in range(4):
            col = _rne_bf16(plsc.load_gather(tbl_ref, [gvs[j] * nch + c]))
            acc = acc + simq[j] * col
        ov = jnp.where(iota == c, jnp.sum(acc), ov)
    ov_ref[...] = ov
    pltpu.sync_copy(ov_ref, o_all.at[pl.ds(off, 16)])


_OUT_TYPE = jax.ShapeDtypeStruct((160,), _F32)

_SCRATCH = [
    pltpu.VMEM((16,), _F32),
    pltpu.VMEM((16,), _F32),
    pltpu.VMEM((_ND * _HWD,), _F32),
    pltpu.VMEM((64,), _I32),
    pltpu.VMEM((80,), _F32),
    pltpu.VMEM((64, 64), _F32),
    pltpu.VMEM((64, 48), _F32),
    pltpu.VMEM((_ND * _VOCAB * 3,), _F32),
    pltpu.VMEM((_ND * _VOCAB * 4,), _F32),
    pltpu.VMEM((_ND * _VOCAB * 2,), _F32),
    pltpu.VMEM((16,), _F32),
    pltpu.SemaphoreType.DMA,
    pltpu.SemaphoreType.DMA,
    pltpu.SemaphoreType.DMA,
]


def _mhn_body(x_r, hw_r, hwt_r, lyr_r, hd_r, mlp_r, emb_r, bias_r,
              o_all,
              xv, hwv, hwtv, gidx, simv,
              rows_h, rows_m, tbl_l, tbl_e, tbl_b,
              ov, sem, sem2, sem3):
    wid = lax.axis_index("s")
    iota = lax.iota(_I32, _L)

    @pl.when(wid < 10)
    def _body():
        pltpu.sync_copy(x_r, xv)
        hw_cp = pltpu.async_copy(hw_r, hwv, sem2)
        hwt_cp = pltpu.async_copy(hwt_r, hwtv, sem3)
        hw_cp.wait()
        hwt_cp.wait()
        sims, gvs = _similarity(xv, hwv, hwtv, iota)
        for j in range(4):
            gidx[pl.ds(16 * j, 16)] = gvs[j]

        for t in range(4):
            @pl.when(wid == t)
            def _head(t=t):
                _wide_group(hd_r, gidx, rows_h, simv, sims, iota, t,
                            ov, o_all, _OFF_HEAD, sem)

        for t in range(3):
            @pl.when(wid == 4 + t)
            def _mlp(t=t):
                _wide_group(mlp_r, gidx, rows_m, simv, sims, iota, t,
                            ov, o_all, _OFF_MLP, sem)

        @pl.when(wid == 7)
        def _layer():
            _narrow(lyr_r, tbl_l, gvs, sims, iota, ov, o_all,
                    _OFF_LAYER, 3, sem)

        @pl.when(wid == 8)
        def _embed():
            _narrow(emb_r, tbl_e, gvs, sims, iota, ov, o_all,
                    _OFF_EMBED, 4, sem)

        @pl.when(wid == 9)
        def _bias():
            _narrow(bias_r, tbl_b, gvs, sims, iota, ov, o_all,
                    _OFF_BIAS, 2, sem)


@functools.cache
def _mhn_kernel():
    mesh = plsc.VectorSubcoreMesh(core_axis_name="c", subcore_axis_name="s",
                                  num_cores=1)
    return pl.kernel(
        _mhn_body, out_type=_OUT_TYPE, mesh=mesh, scratch_types=_SCRATCH,
        compiler_params=pltpu.CompilerParams(needs_layout_passes=False,
                                             use_tc_tiling_on_sc=False))


def kernel(x, hw, hw_table, emb_layer, emb_head, emb_mlp, emb_embed, emb_bias):
    xb = jnp.broadcast_to(x[0, :1], (16,))
    hwp = jnp.zeros((16,), _F32).at[1:11].set(hw)
    o = _mhn_kernel()(
        xb,
        hwp,
        hw_table.reshape(-1),
        emb_layer.reshape(-1),
        emb_head.reshape(-1, 64),
        emb_mlp.reshape(-1, 48),
        emb_embed.reshape(-1),
        emb_bias.reshape(-1),
    )
    return (o[_OFF_LAYER:_OFF_LAYER + 3],
            o[_OFF_HEAD:_OFF_HEAD + 64].reshape(16, 4),
            o[_OFF_MLP:_OFF_MLP + 48].reshape(16, 3),
            o[_OFF_EMBED:_OFF_EMBED + 4],
            o[_OFF_BIAS:_OFF_BIAS + 2])


# speedup vs baseline: 1.0321x; 1.0321x over previous
import functools

import jax
import jax.numpy as jnp
from jax import lax
from jax.experimental import pallas as pl
from jax.experimental.pallas import tpu as pltpu
from jax.experimental.pallas import tpu_sc as plsc

_ND = 50
_HWD = 10
_VOCAB = 101
_L = 16
_F32 = jnp.float32
_I32 = jnp.int32

_OFF_HW = 17
_OFF_HWT = 32
_META = 532

_OFF_LAYER, _OFF_HEAD, _OFF_MLP, _OFF_EMBED, _OFF_BIAS = 0, 16, 80, 128, 144

_NROW = 56


def _full(v):
    return jnp.full((_L,), v, _I32)


def _rne_bf16(v):
    b = plsc.bitcast(v, _I32)
    r = b + jnp.int32(0x7FFF) + ((b >> 16) & 1)
    return plsc.bitcast(r & jnp.int32(-65536), _F32)


def _similarity(metav, iota):
    dcl = [jnp.minimum(iota + 16 * j, _ND - 1) for j in range(4)]

    def dot_step(k, accs):
        hwk = _rne_bf16(plsc.load_gather(metav, [iota * 0 + (_OFF_HW + k)]))
        return tuple(
            accs[j] + hwk * _rne_bf16(
                plsc.load_gather(metav, [dcl[j] * _HWD + (_OFF_HWT + k)]))
            for j in range(4)
        )

    zero = jnp.zeros((_L,), _F32)
    accs = lax.fori_loop(0, _HWD, dot_step, (zero, zero, zero, zero))
    scale = jnp.float32(1.0 / (_HWD ** 0.5))
    zs = [a * scale for a in accs]
    m = jnp.max(jnp.maximum(jnp.maximum(zs[0], zs[1]),
                            jnp.maximum(zs[2], zs[3])))
    es = [jnp.where(iota + 16 * j < _ND, jnp.exp(zs[j] - m),
                    jnp.float32(0.0)) for j in range(4)]
    s = jnp.sum(es[0] + es[1] + es[2] + es[3])
    rv = jnp.full((_L,), 1.0, _F32) / jnp.broadcast_to(s, (_L,))
    return [e * rv for e in es]


def _wide(tbl_hbm, metav, gidx, rows_ref, sim_ref, iota, nvec,
          ov_ref, o_all, off, sem):
    cp = pltpu.async_copy(tbl_hbm.at[gidx.at[pl.ds(0, _NROW)]], rows_ref, sem)
    sims = _similarity(metav, iota)
    for j in range(4):
        sim_ref[pl.ds(16 + 16 * j, 16)] = sims[j]
    cp.wait()

    def step(d, accs):
        sd = _rne_bf16(plsc.load_gather(sim_ref, [iota * 0 + (16 + d)]))
        return tuple(
            accs[t] + sd * _rne_bf16(
                plsc.load_gather(rows_ref, [_full(0) + d, iota + 16 * t]))
            for t in range(nvec)
        )

    accs = lax.fori_loop(0, _ND, step,
                         tuple(jnp.zeros((_L,), _F32) for _ in range(nvec)))
    for t in range(nvec):
        ov_ref[pl.ds(16 * t, 16)] = accs[t]
    pltpu.sync_copy(ov_ref.at[pl.ds(0, 16 * nvec)],
                    o_all.at[pl.ds(off, 16 * nvec)])


def _narrow_all(tbls_hbm, tbl_refs, metav, iota, ov_ref, o_all, sem):
    cps = [pltpu.async_copy(h, r, sem) for h, r in zip(tbls_hbm, tbl_refs)]
    xv = metav[pl.ds(0, 16)]
    idxb = (xv * 101.0).astype(_I32)
    gvs = [jnp.minimum(iota + 16 * j, _ND - 1) * _VOCAB + idxb
           for j in range(4)]
    sims = _similarity(metav, iota)
    simq = [_rne_bf16(sj) for sj in sims]
    for cp in cps:
        cp.wait()
    for tbl_ref, off, nch in zip(tbl_refs,
                                 (_OFF_LAYER, _OFF_EMBED, _OFF_BIAS),
                                 (3, 4, 2)):
        ov = jnp.zeros((_L,), _F32)
        for c in range(nch):
            acc = jnp.zeros((_L,), _F32)
            for j in range(4):
                col = _rne_bf16(
                    plsc.load_gather(tbl_ref, [gvs[j] * nch + c]))
                acc = acc + simq[j] * col
            ov = jnp.where(iota == c, jnp.sum(acc), ov)
        ov_ref[pl.ds(0, 16)] = ov
        pltpu.sync_copy(ov_ref.at[pl.ds(0, 16)], o_all.at[pl.ds(off, 16)])


_OUT_TYPE = jax.ShapeDtypeStruct((160,), _F32)

_SCRATCH = [
    pltpu.VMEM((_META,), _F32),
    pltpu.VMEM((64,), _I32),
    pltpu.VMEM((80,), _F32),
    pltpu.VMEM((_NROW, 64), _F32),
    pltpu.VMEM((_NROW, 48), _F32),
    pltpu.VMEM((_ND * _VOCAB * 3,), _F32),
    pltpu.VMEM((_ND * _VOCAB * 4,), _F32),
    pltpu.VMEM((_ND * _VOCAB * 2,), _F32),
    pltpu.VMEM((64,), _F32),
    pltpu.SemaphoreType.DMA,
]


def _mhn_body(meta_r, lyr_r, hd_r, mlp_r, emb_r, bias_r,
              o_all,
              metav, gidx, simv, rows_h, rows_m, tbl_l, tbl_e, tbl_b,
              ov, sem):
    wid = lax.axis_index("s")
    iota = lax.iota(_I32, _L)

    @pl.when(wid < 3)
    def _body():
        pltpu.sync_copy(meta_r, metav)
        xv = metav[pl.ds(0, 16)]
        idxb = (xv * 101.0).astype(_I32)
        for j in range(4):
            gidx[pl.ds(16 * j, 16)] = (
                jnp.minimum(iota + 16 * j, _ND - 1) * _VOCAB + idxb)

        @pl.when(wid == 0)
        def _head():
            _wide(hd_r, metav, gidx, rows_h, simv, iota, 4,
                  ov, o_all, _OFF_HEAD, sem)

        @pl.when(wid == 1)
        def _mlp():
            _wide(mlp_r, metav, gidx, rows_m, simv, iota, 3,
                  ov, o_all, _OFF_MLP, sem)

        @pl.when(wid == 2)
        def _nar():
            _narrow_all((lyr_r, emb_r, bias_r), (tbl_l, tbl_e, tbl_b),
                        metav, iota, ov, o_all, sem)


@functools.cache
def _mhn_kernel():
    mesh = plsc.VectorSubcoreMesh(core_axis_name="c", subcore_axis_name="s",
                                  num_cores=1)
    return pl.kernel(
        _mhn_body, out_type=_OUT_TYPE, mesh=mesh, scratch_types=_SCRATCH,
        compiler_params=pltpu.CompilerParams(needs_layout_passes=False,
                                             use_tc_tiling_on_sc=False))


def kernel(x, hw, hw_table, emb_layer, emb_head, emb_mlp, emb_embed, emb_bias):
    meta = jnp.concatenate([
        jnp.broadcast_to(x[0, :1], (16,)),
        jnp.zeros((1,), _F32), hw, jnp.zeros((5,), _F32),
        hw_table.reshape(-1),
    ])
    o = _mhn_kernel()(
        meta,
        emb_layer.reshape(-1),
        emb_head.reshape(-1, 64),
        emb_mlp.reshape(-1, 48),
        emb_embed.reshape(-1),
        emb_bias.reshape(-1),
    )
    return (o[_OFF_LAYER:_OFF_LAYER + 3],
            o[_OFF_HEAD:_OFF_HEAD + 64].reshape(16, 4),
            o[_OFF_MLP:_OFF_MLP + 48].reshape(16, 3),
            o[_OFF_EMBED:_OFF_EMBED + 4],
            o[_OFF_BIAS:_OFF_BIAS + 2])


# speedup vs baseline: 1.1974x; 1.1601x over previous
import functools

import jax
import jax.numpy as jnp
from jax import lax
from jax.experimental import pallas as pl
from jax.experimental.pallas import tpu as pltpu
from jax.experimental.pallas import tpu_sc as plsc

_ND = 50
_HWD = 10
_VOCAB = 101
_L = 16
_F32 = jnp.float32
_I32 = jnp.int32

_OFF_HW = 17
_OFF_HWT = 32
_META = 532

_OFF_HEAD, _OFF_MLP, _OFF_NAR = 0, 64, 112

_NROW = 56


def _full(v):
    return jnp.full((_L,), v, _I32)


def _rne_bf16(v):
    b = plsc.bitcast(v, _I32)
    r = b + jnp.int32(0x7FFF) + ((b >> 16) & 1)
    return plsc.bitcast(r & jnp.int32(-65536), _F32)


def _similarity(metav, iota):
    dcl = [jnp.minimum(iota + 16 * j, _ND - 1) for j in range(4)]

    def dot_step(k, accs):
        hwk = _rne_bf16(plsc.load_gather(metav, [iota * 0 + (_OFF_HW + k)]))
        return tuple(
            accs[j] + hwk * _rne_bf16(
                plsc.load_gather(metav, [dcl[j] * _HWD + (_OFF_HWT + k)]))
            for j in range(4)
        )

    zero = jnp.zeros((_L,), _F32)
    accs = lax.fori_loop(0, _HWD, dot_step, (zero, zero, zero, zero))
    scale = jnp.float32(1.0 / (_HWD ** 0.5))
    zs = [a * scale for a in accs]
    m = jnp.max(jnp.maximum(jnp.maximum(zs[0], zs[1]),
                            jnp.maximum(zs[2], zs[3])))
    es = [jnp.where(iota + 16 * j < _ND, jnp.exp(zs[j] - m),
                    jnp.float32(0.0)) for j in range(4)]
    s = jnp.sum(es[0] + es[1] + es[2] + es[3])
    rv = jnp.full((_L,), 1.0, _F32) / jnp.broadcast_to(s, (_L,))
    return [e * rv for e in es]


def _wide(tbl_hbm, metav, gidx, rows_ref, sim_ref, iota, nvec,
          ov_ref, o_all, off, sem):
    cp = pltpu.async_copy(tbl_hbm.at[gidx.at[pl.ds(0, _NROW)]], rows_ref, sem)
    sims = _similarity(metav, iota)
    for j in range(4):
        sim_ref[pl.ds(16 + 16 * j, 16)] = sims[j]
    cp.wait()

    def step(d, accs):
        sd = _rne_bf16(plsc.load_gather(sim_ref, [iota * 0 + (16 + d)]))
        return tuple(
            accs[t] + sd * _rne_bf16(
                plsc.load_gather(rows_ref, [_full(0) + d, iota + 16 * t]))
            for t in range(nvec)
        )

    accs = lax.fori_loop(0, _ND, step,
                         tuple(jnp.zeros((_L,), _F32) for _ in range(nvec)))
    for t in range(nvec):
        ov_ref[pl.ds(16 * t, 16)] = accs[t]
    pltpu.sync_copy(ov_ref.at[pl.ds(0, 16 * nvec)],
                    o_all.at[pl.ds(off, 16 * nvec)])


_OUT_TYPE = jax.ShapeDtypeStruct((160,), _F32)

_SCRATCH = [
    pltpu.VMEM((_META,), _F32),
    pltpu.VMEM((64,), _I32),
    pltpu.VMEM((80,), _F32),
    pltpu.VMEM((_NROW, 64), _F32),
    pltpu.VMEM((_NROW, 48), _F32),
    pltpu.VMEM((_NROW, 16), _F32),
    pltpu.VMEM((64,), _F32),
    pltpu.SemaphoreType.DMA,
]


def _mhn_body(meta_r, hd_r, mlp_r, nar_r,
              o_all,
              metav, gidx, simv, rows_h, rows_m, rows_n,
              ov, sem):
    wid = lax.axis_index("s")
    iota = lax.iota(_I32, _L)

    @pl.when(wid < 3)
    def _body():
        pltpu.sync_copy(meta_r, metav)
        xv = metav[pl.ds(0, 16)]
        idxb = (xv * 101.0).astype(_I32)
        for j in range(4):
            gidx[pl.ds(16 * j, 16)] = (
                jnp.minimum(iota + 16 * j, _ND - 1) * _VOCAB + idxb)

        @pl.when(wid == 0)
        def _head():
            _wide(hd_r, metav, gidx, rows_h, simv, iota, 4,
                  ov, o_all, _OFF_HEAD, sem)

        @pl.when(wid == 1)
        def _mlp():
            _wide(mlp_r, metav, gidx, rows_m, simv, iota, 3,
                  ov, o_all, _OFF_MLP, sem)

        @pl.when(wid == 2)
        def _nar():
            _wide(nar_r, metav, gidx, rows_n, simv, iota, 1,
                  ov, o_all, _OFF_NAR, sem)


@functools.cache
def _mhn_kernel():
    mesh = plsc.VectorSubcoreMesh(core_axis_name="c", subcore_axis_name="s",
                                  num_cores=1)
    return pl.kernel(
        _mhn_body, out_type=_OUT_TYPE, mesh=mesh, scratch_types=_SCRATCH,
        compiler_params=pltpu.CompilerParams(needs_layout_passes=False,
                                             use_tc_tiling_on_sc=False))


def kernel(x, hw, hw_table, emb_layer, emb_head, emb_mlp, emb_embed, emb_bias):
    meta = jnp.concatenate([
        jnp.broadcast_to(x[0, :1], (16,)),
        jnp.zeros((1,), _F32), hw, jnp.zeros((5,), _F32),
        hw_table.reshape(-1),
    ])
    nrows = _ND * _VOCAB
    nar = jnp.concatenate([
        emb_layer.reshape(nrows, 3),
        emb_embed.reshape(nrows, 4),
        emb_bias.reshape(nrows, 2),
        jnp.zeros((nrows, 7), _F32),
    ], axis=1)
    o = _mhn_kernel()(
        meta,
        emb_head.reshape(-1, 64),
        emb_mlp.reshape(-1, 48),
        nar,
    )
    return (o[_OFF_NAR:_OFF_NAR + 3],
            o[_OFF_HEAD:_OFF_HEAD + 64].reshape(16, 4),
            o[_OFF_MLP:_OFF_MLP + 48].reshape(16, 3),
            o[_OFF_NAR + 3:_OFF_NAR + 7],
            o[_OFF_NAR + 7:_OFF_NAR + 9])
